# Initial kernel scaffold; baseline (speedup 1.0000x reference)
import jax
import jax.numpy as jnp
from jax.experimental import pallas as pl


def kernel(image):
    raise NotImplementedError("write your pallas kernel here")


# speedup vs baseline: 26.8588x; 26.8588x over previous
import jax
import jax.numpy as jnp
from jax import lax
from jax.experimental import pallas as pl
from jax.experimental.pallas import tpu as pltpu

_R, _C = 128, 32768
_N = _R * _C
_POS = 0.75 * (_N - 1)
_K = int(_POS)
_FRAC = _POS - _K
_CH = 8
_NCH = _R // _CH
_MIN32 = -2147483648
_MAX32 = 2147483647


def _sortable_key(x):
    bits = lax.bitcast_convert_type(x, jnp.int32)
    u = jnp.where(bits >= 0, bits | jnp.int32(_MIN32), ~bits)
    return u ^ jnp.int32(_MIN32)


def _key_to_float(k):
    u = k ^ jnp.int32(_MIN32)
    bits = jnp.where(u < 0, u & jnp.int32(_MAX32), ~u)
    return lax.bitcast_convert_type(bits, jnp.float32)


def _body(x_ref, out_ref, key_ref):
    def fill(ci, s):
        xa = x_ref[pl.ds(ci * _CH, _CH), :]
        key_ref[pl.ds(ci * _CH, _CH), :] = _sortable_key(xa)
        return s + jnp.sum(xa)

    total = lax.fori_loop(0, _NCH, fill, jnp.float32(0.0))
    m = total / _N

    def count_below(tk):
        def chunk(ci, acc):
            ka = key_ref[pl.ds(ci * _CH, _CH), :]
            return acc + jnp.sum((ka < tk).astype(jnp.int32))
        return lax.fori_loop(0, _NCH, chunk, jnp.int32(0))

    def step(i, vp):
        t = vp | (jnp.int32(1) << (jnp.int32(31) - i))
        cnt = count_below(t ^ jnp.int32(_MIN32))
        return jnp.where(cnt > _K, vp, t)

    vp = lax.fori_loop(0, 32, step, jnp.int32(0), unroll=False)
    key_k = vp ^ jnp.int32(_MIN32)

    def chunk_le(ci, acc):
        ka = key_ref[pl.ds(ci * _CH, _CH), :]
        return acc + jnp.sum((ka <= key_k).astype(jnp.int32))

    cnt_le = lax.fori_loop(0, _NCH, chunk_le, jnp.int32(0))

    def chunk_above(ci, acc):
        ka = key_ref[pl.ds(ci * _CH, _CH), :]
        return jnp.minimum(acc, jnp.min(jnp.where(ka > key_k, ka, jnp.int32(_MAX32))))

    min_above = lax.fori_loop(0, _NCH, chunk_above, jnp.int32(_MAX32))
    key_k1 = jnp.where(cnt_le >= _K + 2, key_k, min_above)

    xk = _key_to_float(key_k)
    xk1 = _key_to_float(key_k1)
    q = xk * (1.0 - _FRAC) + xk1 * _FRAC

    def mask(ci, carry):
        xa = x_ref[pl.ds(ci * _CH, _CH), :]
        out_ref[pl.ds(ci * _CH, _CH), :] = jnp.where(xa > q, m, xa)
        return carry

    lax.fori_loop(0, _NCH, mask, jnp.int32(0))


def _call(x, interpret=False):
    return pl.pallas_call(
        _body,
        out_shape=jax.ShapeDtypeStruct((_R, _C), jnp.float32),
        scratch_shapes=[pltpu.VMEM((_R, _C), jnp.int32)],
        interpret=interpret,
    )(x)


@jax.jit
def kernel(image):
    return _call(image)
